# Initial kernel scaffold; baseline (speedup 1.0000x reference)
import jax
import jax.numpy as jnp
from jax.experimental import pallas as pl


def kernel(x, edge_index, W1a, b1a, W2a, b2a, W1b, b1b, W2b, b2b):
    raise NotImplementedError("write your pallas kernel here")


# speedup vs baseline: 6.4782x; 6.4782x over previous
import functools

import jax
import jax.numpy as jnp
from jax import lax
from jax.experimental import pallas as pl
from jax.experimental.pallas import tpu as pltpu
from jax.experimental.pallas import tpu_sc as plsc

N = 10000
E = 320000
D = 128

NC = 2
NS = 16
NW = NC * NS

CH = 128
NCHUNK = E // CH
ITERS = -(-NCHUNK // NW)
NPAD = 10240
RPT = NPAD // NS


def _seg_sum_body(x_hbm, edges_hbm, out_hbm, src_v, dst_v, rows_v, acc_sh, sem):
    cid = lax.axis_index("c")
    sid = lax.axis_index("s")
    wid = sid * NC + cid

    def zbody(i, _):
        r = i // (D // 16)
        c = (i % (D // 16)) * 16
        rows_v[r, pl.ds(c, 16)] = jnp.zeros((16,), jnp.float32)
        return 0

    lax.fori_loop(0, CH * (D // 16), zbody, 0)

    base = sid * RPT
    for j in range(RPT // CH):
        pltpu.sync_copy(rows_v, acc_sh.at[pl.ds(base + j * CH, CH)])

    plsc.subcore_barrier()

    def ebody(i, _):
        c = i * NW + wid

        @pl.when(c < NCHUNK)
        def _():
            off = c * CH
            pltpu.sync_copy(edges_hbm.at[0, pl.ds(off, CH)], src_v)
            pltpu.sync_copy(edges_hbm.at[1, pl.ds(off, CH)], dst_v)
            pltpu.async_copy(x_hbm.at[src_v], rows_v, sem).wait()
            pltpu.sync_copy(rows_v, acc_sh.at[dst_v], add=True)

        return 0

    lax.fori_loop(0, ITERS, ebody, 0)

    plsc.subcore_barrier()

    pltpu.sync_copy(acc_sh.at[pl.ds(base, RPT)], out_hbm.at[cid, pl.ds(base, RPT)])


@jax.jit
def _seg_sum(x, edges):
    mesh = plsc.VectorSubcoreMesh(core_axis_name="c", subcore_axis_name="s")
    return pl.kernel(
        _seg_sum_body,
        out_type=jax.ShapeDtypeStruct((NC, NPAD, D), jnp.float32),
        mesh=mesh,
        scratch_types=[
            pltpu.VMEM((CH,), jnp.int32),
            pltpu.VMEM((CH,), jnp.int32),
            pltpu.VMEM((CH, D), jnp.float32),
            pltpu.VMEM_SHARED((NPAD, D), jnp.float32),
            pltpu.SemaphoreType.DMA,
        ],
    )(x, edges)


BR = 1000


def _mlp_body(x_ref, p_ref, w1_ref, b1_ref, w2_ref, b2_ref, o_ref, *, final):
    h = x_ref[...] + p_ref[0] + p_ref[1]
    t = jnp.dot(h, w1_ref[...], preferred_element_type=jnp.float32) + b1_ref[...]
    t = jnp.maximum(t, 0.0)
    o = jnp.dot(t, w2_ref[...], preferred_element_type=jnp.float32) + b2_ref[...]
    if final:
        m = jnp.max(o, axis=1, keepdims=True)
        o = o - m
        o_ref[...] = o - jnp.log(jnp.sum(jnp.exp(o), axis=1, keepdims=True))
    else:
        o_ref[...] = jnp.maximum(o, 0.0)


def _mlp(x, p, w1, b1, w2, b2, final):
    grid = (N // BR,)
    return pl.pallas_call(
        functools.partial(_mlp_body, final=final),
        grid=grid,
        in_specs=[
            pl.BlockSpec((BR, D), lambda i: (i, 0)),
            pl.BlockSpec((NC, BR, D), lambda i: (0, i, 0)),
            pl.BlockSpec((D, D), lambda i: (0, 0)),
            pl.BlockSpec((1, D), lambda i: (0, 0)),
            pl.BlockSpec((D, D), lambda i: (0, 0)),
            pl.BlockSpec((1, D), lambda i: (0, 0)),
        ],
        out_specs=pl.BlockSpec((BR, D), lambda i: (i, 0)),
        out_shape=jax.ShapeDtypeStruct((N, D), jnp.float32),
    )(x, p, w1, b1, w2, b2)


def kernel(x, edge_index, W1a, b1a, W2a, b2a, W1b, b1b, W2b, b2b):
    p1 = _seg_sum(x, edge_index)
    h = _mlp(x, p1, W1a, b1a.reshape(1, D), W2a, b2a.reshape(1, D), final=False)
    p2 = _seg_sum(h, edge_index)
    return _mlp(h, p2, W1b, b1b.reshape(1, D), W2b, b2b.reshape(1, D), final=True)
